# baseline (speedup 1.0000x reference)
import functools

import jax
import jax.numpy as jnp
from jax import lax
from jax.experimental import pallas as pl
from jax.experimental.pallas import tpu as pltpu
from jax.experimental.pallas import tpu_sc as plsc

B = 256
BS = 19
HW = BS * BS
HWP = 368
MAXM = HW
NW = 32
GPW = B // NW
NCHUNK = HWP // 16


def _body(flat_hbm, pad_hbm, r_hbm, c_hbm, cp_hbm, pc_hbm, mv_hbm, ko_hbm,
          hist_ref,
          board_out, mc_out, pc_out, ko_out, pl_out, sc_out,
          src_rows, b2, r_vm, c_vm, cp_vm, pc_vm, mv_vm, ko_vm,
          mcw, pcw, plw, scw, sem):
  wid = lax.axis_index("s") * 2 + lax.axis_index("c")
  base = wid * GPW

  pltpu.sync_copy(flat_hbm.at[pl.ds(base, GPW)], src_rows)
  pltpu.sync_copy(pad_hbm.at[pl.ds(base, GPW)], b2)
  pltpu.sync_copy(r_hbm.at[pl.ds(base, GPW)], r_vm.at[pl.ds(0, GPW)])
  pltpu.sync_copy(c_hbm.at[pl.ds(base, GPW)], c_vm.at[pl.ds(0, GPW)])
  pltpu.sync_copy(cp_hbm.at[pl.ds(base, GPW)], cp_vm.at[pl.ds(0, GPW)])
  pltpu.sync_copy(pc_hbm.at[pl.ds(base, GPW)], pc_vm.at[pl.ds(0, GPW)])
  pltpu.sync_copy(mv_hbm.at[pl.ds(base, GPW)], mv_vm.at[pl.ds(0, GPW)])
  pltpu.sync_copy(ko_hbm.at[pl.ds(2 * base, 16)], ko_vm)

  lane = lax.iota(jnp.int32, 16)
  g8 = lane < GPW
  r = r_vm[...]
  c = c_vm[...]
  cp = cp_vm[...]
  pc = pc_vm[...]
  mv = mv_vm[...]

  is_pass = (r < 0) | (c < 0)
  play = jnp.logical_not(is_pass) & g8
  rr = jnp.clip(r, 0, BS - 1)
  cc = jnp.clip(c, 0, BS - 1)
  cell = rr * BS + cc

  mvc = jnp.clip(mv, 0, MAXM - 1)
  hist_cps = []
  for g in range(GPW):
    hist_cps.append(
        pltpu.async_copy(src_rows.at[g], hist_ref.at[base + g, mvc[g]], sem))

  plsc.store_scatter(b2, [lane, cell], cp.astype(jnp.float32), mask=play)

  scores16 = jnp.zeros((16,), jnp.float32)
  for g in range(GPW):
    c0 = jnp.zeros((16,), jnp.int32)
    c1 = jnp.zeros((16,), jnp.int32)
    for j in range(NCHUNK):
      x = b2[g, pl.ds(16 * j, 16)]
      c0 = c0 + plsc.all_reduce_population_count(x == 0.0)
      c1 = c1 + plsc.all_reduce_population_count(x == 1.0)
    scores16 = jnp.where(lane == 2 * g, c0.astype(jnp.float32), scores16)
    scores16 = jnp.where(lane == 2 * g + 1, c1.astype(jnp.float32), scores16)
  scw[...] = scores16

  mcw[...] = mv + 1
  pcw[...] = jnp.where(is_pass, pc + 1, 0)
  plw[...] = cp ^ 1
  plsc.store_scatter(ko_vm, [2 * lane], jnp.full((16,), -1, jnp.int32),
                     mask=play)
  plsc.store_scatter(ko_vm, [2 * lane + 1], jnp.full((16,), -1, jnp.int32),
                     mask=play)

  pltpu.sync_copy(b2, board_out.at[pl.ds(base, GPW)])
  pltpu.sync_copy(mcw.at[pl.ds(0, GPW)], mc_out.at[pl.ds(base, GPW)])
  pltpu.sync_copy(pcw.at[pl.ds(0, GPW)], pc_out.at[pl.ds(base, GPW)])
  pltpu.sync_copy(plw.at[pl.ds(0, GPW)], pl_out.at[pl.ds(base, GPW)])
  pltpu.sync_copy(ko_vm, ko_out.at[pl.ds(2 * base, 16)])
  pltpu.sync_copy(scw, sc_out.at[pl.ds(2 * base, 16)])
  for hist_cp in hist_cps:
    hist_cp.wait()


@functools.cache
def _make_sc_step():
  mesh = plsc.VectorSubcoreMesh(core_axis_name="c", subcore_axis_name="s",
                                num_cores=2, num_subcores=16)
  return pl.kernel(
      _body,
      out_type=(
          jax.ShapeDtypeStruct((B, HWP), jnp.float32),
          jax.ShapeDtypeStruct((B,), jnp.int32),
          jax.ShapeDtypeStruct((B,), jnp.int32),
          jax.ShapeDtypeStruct((2 * B,), jnp.int32),
          jax.ShapeDtypeStruct((B,), jnp.int32),
          jax.ShapeDtypeStruct((2 * B,), jnp.float32),
      ),
      mesh=mesh,
      compiler_params=pltpu.CompilerParams(needs_layout_passes=False),
      scratch_types=(
          pltpu.VMEM((GPW, HW), jnp.float32),
          pltpu.VMEM((GPW, HWP), jnp.float32),
          pltpu.VMEM((16,), jnp.int32),
          pltpu.VMEM((16,), jnp.int32),
          pltpu.VMEM((16,), jnp.int32),
          pltpu.VMEM((16,), jnp.int32),
          pltpu.VMEM((16,), jnp.int32),
          pltpu.VMEM((16,), jnp.int32),
          pltpu.VMEM((16,), jnp.int32),
          pltpu.VMEM((16,), jnp.int32),
          pltpu.VMEM((16,), jnp.int32),
          pltpu.VMEM((16,), jnp.float32),
          pltpu.SemaphoreType.DMA,
      ),
  )


def kernel(positions, board, current_player, ko_points, pass_count,
           board_history, move_count):
  flat = board.reshape(B, HW)
  pad = jnp.pad(flat, ((0, 0), (0, HWP - HW)), constant_values=-1.0)
  r = positions[:, 0]
  c = positions[:, 1]
  ko_flat = ko_points.reshape(2 * B)

  hist_ref = jax.new_ref(board_history)

  board_pad, mc, pco, koo, plo, sco = _make_sc_step()(
      flat, pad, r, c, current_player, pass_count, move_count, ko_flat,
      hist_ref)

  new_board = board_pad[:, :HW].reshape(B, BS, BS)
  new_history = hist_ref[...]
  return (new_board, new_history, mc, pco, koo.reshape(B, 2), plo,
          sco.reshape(B, 2))
